# baseline (device time: 165213 ns/iter reference)
import functools

import jax
import jax.numpy as jnp
from jax import lax
from jax.experimental import pallas as pl
from jax.experimental.pallas import tpu as pltpu

N_DEV = 4
NB = 8


def kernel(x, w_mat):
    m_global, k_per = x.shape
    _, n = w_mat.shape
    m_per = m_global // N_DEV
    h = m_per // 2
    nblk = n // NB

    def body(
        x_hbm,
        w_hbm,
        out_ref,
        comm_r,
        comm_l,
        stage_ra,
        stage_la,
        stage_rb,
        stage_lb,
        xc_hi,
        xc_lo,
        xland,
        w_hi,
        w_lo,
        wland,
        amax_src_ref,
        amax_comm_ref,
        send_r,
        recv_r,
        send_l,
        recv_l,
        credit_r,
        credit_l,
        xdma_sem,
        wdma_sem,
        amax_send_sems,
        amax_recv_sems,
    ):
        my = lax.axis_index("i")
        left = lax.rem(my + N_DEV - 1, N_DEV)
        right = lax.rem(my + 1, N_DEV)

        TOP, BOT = 0, 1

        def load(owner, half):
            cp = pltpu.make_async_copy(
                x_hbm.at[pl.ds(owner * m_per + half * h, h), :],
                xland,
                xdma_sem,
            )
            cp.start()
            return cp

        def xconv(slot):
            f = xland[...]
            hi = f.astype(jnp.bfloat16)
            xc_hi[slot] = hi
            xc_lo[slot] = (f - hi.astype(jnp.float32)).astype(jnp.bfloat16)

        def load2(owner_top, owner_bot):
            load(owner_top, TOP).wait()
            xconv(0)
            load(owner_bot, BOT).wait()
            xconv(1)

        def wconv(b):
            cols = pl.ds(b * nblk, nblk)
            for piece in range(2):
                rows = pl.ds(piece * (k_per // 2), k_per // 2)
                cp = pltpu.make_async_copy(
                    w_hbm.at[rows, cols], wland, wdma_sem
                )
                cp.start()
                cp.wait()
                f = wland[...]
                hi = f.astype(jnp.bfloat16)
                w_hi[rows, cols] = hi
                w_lo[rows, cols] = (
                    f - hi.astype(jnp.float32)
                ).astype(jnp.bfloat16)

        c0 = load(lax.rem(my + 3, N_DEV), TOP)

        barrier_sem = pltpu.get_barrier_semaphore()
        for nbr in (left, right):
            pl.semaphore_signal(
                barrier_sem,
                inc=1,
                device_id=(nbr,),
                device_id_type=pl.DeviceIdType.MESH,
            )
        pl.semaphore_wait(barrier_sem, 2)

        def mmb(dst, slot, b):
            cols = pl.ds(b * nblk, nblk)
            xh, xl = xc_hi[slot], xc_lo[slot]
            wh, wl = w_hi[:, cols], w_lo[:, cols]
            dst[...] = jnp.dot(xh, wh, preferred_element_type=jnp.float32)
            dst[...] = dst[...] + jnp.dot(
                xh, wl, preferred_element_type=jnp.float32
            )
            dst[...] = dst[...] + jnp.dot(
                xl, wh, preferred_element_type=jnp.float32
            )

        def rdma(src, dst, ssem, rsem, dev):
            return pltpu.make_async_remote_copy(
                src_ref=src,
                dst_ref=dst,
                send_sem=ssem,
                recv_sem=rsem,
                device_id=(dev,),
                device_id_type=pl.DeviceIdType.MESH,
            )

        r0, l0 = [], []

        def send0_r(b):
            mmb(stage_ra.at[b], 0, b)
            rb = rdma(
                stage_ra.at[b], comm_r.at[0, b],
                send_r.at[0, b], recv_r.at[0, b], right,
            )
            rb.start()
            r0.append(rb)

        def send0_l(b):
            mmb(stage_la.at[b], 1, b)
            lb = rdma(
                stage_la.at[b], comm_l.at[0, b],
                send_l.at[0, b], recv_l.at[0, b], left,
            )
            lb.start()
            l0.append(lb)

        c0.wait()
        xconv(0)
        c1 = load(lax.rem(my + 1, N_DEV), BOT)
        wconv(0)
        send0_r(0)
        c1.wait()
        xconv(1)
        send0_l(0)
        for b in range(1, NB):
            wconv(b)
            send0_r(b)
            send0_l(b)

        load2(lax.rem(my + 2, N_DEV), lax.rem(my + 2, N_DEV))
        for b in range(NB):
            mmb(stage_rb.at[b], 0, b)
            mmb(stage_lb.at[b], 1, b)

        r1, l1 = [], []
        for b in range(NB):
            r0[b].wait_recv()
            stage_rb[b] = stage_rb[b] + comm_r[0, b]
            rb = rdma(
                stage_rb.at[b], comm_r.at[1, b],
                send_r.at[1, b], recv_r.at[1, b], right,
            )
            rb.start()
            r1.append(rb)
            l0[b].wait_recv()
            stage_lb[b] = stage_lb[b] + comm_l[0, b]
            lb = rdma(
                stage_lb.at[b], comm_l.at[1, b],
                send_l.at[1, b], recv_l.at[1, b], left,
            )
            lb.start()
            l1.append(lb)
        pl.semaphore_signal(
            credit_r, inc=1, device_id=(left,),
            device_id_type=pl.DeviceIdType.MESH,
        )
        pl.semaphore_signal(
            credit_l, inc=1, device_id=(right,),
            device_id_type=pl.DeviceIdType.MESH,
        )

        for b in range(NB):
            r0[b].wait_send()
            l0[b].wait_send()
        load2(lax.rem(my + 1, N_DEV), lax.rem(my + 3, N_DEV))
        for b in range(NB):
            mmb(stage_ra.at[b], 0, b)
            mmb(stage_la.at[b], 1, b)

        pl.semaphore_wait(credit_r, 1)
        pl.semaphore_wait(credit_l, 1)
        r2, l2 = [], []
        for b in range(NB):
            r1[b].wait_recv()
            stage_ra[b] = stage_ra[b] + comm_r[1, b]
            rb = rdma(
                stage_ra.at[b], comm_r.at[0, b],
                send_r.at[2, b], recv_r.at[0, b], right,
            )
            rb.start()
            r2.append(rb)
            l1[b].wait_recv()
            stage_la[b] = stage_la[b] + comm_l[1, b]
            lb = rdma(
                stage_la.at[b], comm_l.at[0, b],
                send_l.at[2, b], recv_l.at[0, b], left,
            )
            lb.start()
            l2.append(lb)

        load2(my, my)
        out_top = out_ref.at[pl.ds(0, h)]
        out_bot = out_ref.at[pl.ds(h, h)]
        for b in range(NB):
            cols = pl.ds(b * nblk, nblk)
            mmb(out_top.at[:, cols], 0, b)
            mmb(out_bot.at[:, cols], 1, b)

        local_max = jnp.float32(0.0)
        for b in range(NB):
            cols = pl.ds(b * nblk, nblk)
            r2[b].wait_recv()
            yt = jnp.maximum(out_top[:, cols] + comm_r[0, b], 0.0)
            out_top[:, cols] = yt
            local_max = jnp.maximum(local_max, jnp.max(yt))
            l2[b].wait_recv()
            yb = jnp.maximum(out_bot[:, cols] + comm_l[0, b], 0.0)
            out_bot[:, cols] = yb
            local_max = jnp.maximum(local_max, jnp.max(yb))

        amax_src_ref[...] = jnp.full((8, 128), local_max, jnp.float32)
        amax_comm_ref[my] = amax_src_ref[...]
        peers = [lax.rem(my + d, N_DEV) for d in (1, 2, 3)]
        sends = []
        for j, q in enumerate(peers):
            s = rdma(
                amax_src_ref,
                amax_comm_ref.at[my],
                amax_send_sems.at[j],
                amax_recv_sems.at[my],
                q,
            )
            s.start()
            sends.append(s)
        for q in peers:
            rdma(
                amax_src_ref,
                amax_comm_ref.at[q],
                amax_send_sems.at[0],
                amax_recv_sems.at[q],
                right,
            ).wait_recv()
        gmax = jnp.max(amax_comm_ref[...])

        scale = gmax / 448.0
        inv_scale = 448.0 / gmax
        for b in range(NB):
            cols = pl.ds(b * nblk, nblk)
            q8 = jnp.minimum(out_ref[:, cols] * inv_scale, 448.0).astype(
                jnp.float8_e4m3fn
            )
            out_ref[:, cols] = q8.astype(jnp.float32) * scale

        for d in sends:
            d.wait_send()
        for b in range(NB):
            r1[b].wait_send()
            l1[b].wait_send()
            r2[b].wait_send()
            l2[b].wait_send()

        @functools.partial(
            pl.run_scoped, second_barrier=pltpu.SemaphoreType.REGULAR
        )
        def _(second_barrier):
            for nbr in (left, right):
                pl.semaphore_signal(
                    second_barrier,
                    inc=1,
                    device_id=(nbr,),
                    device_id_type=pl.DeviceIdType.MESH,
                )
            pl.semaphore_wait(second_barrier, 2)

    return pl.pallas_call(
        body,
        out_shape=jax.ShapeDtypeStruct((m_per, n), jnp.float32),
        in_specs=[
            pl.BlockSpec(memory_space=pl.ANY),
            pl.BlockSpec(memory_space=pl.ANY),
        ],
        out_specs=pl.BlockSpec(memory_space=pltpu.VMEM),
        scratch_shapes=[
            pltpu.VMEM((2, NB, h, n // NB), jnp.float32),
            pltpu.VMEM((2, NB, h, n // NB), jnp.float32),
            pltpu.VMEM((NB, h, n // NB), jnp.float32),
            pltpu.VMEM((NB, h, n // NB), jnp.float32),
            pltpu.VMEM((NB, h, n // NB), jnp.float32),
            pltpu.VMEM((NB, h, n // NB), jnp.float32),
            pltpu.VMEM((2, h, k_per), jnp.bfloat16),
            pltpu.VMEM((2, h, k_per), jnp.bfloat16),
            pltpu.VMEM((h, k_per), jnp.float32),
            pltpu.VMEM((k_per, n), jnp.bfloat16),
            pltpu.VMEM((k_per, n), jnp.bfloat16),
            pltpu.VMEM((k_per // 2, n // NB), jnp.float32),
            pltpu.VMEM((8, 128), jnp.float32),
            pltpu.VMEM((N_DEV, 8, 128), jnp.float32),
            pltpu.SemaphoreType.DMA((3, NB)),
            pltpu.SemaphoreType.DMA((2, NB)),
            pltpu.SemaphoreType.DMA((3, NB)),
            pltpu.SemaphoreType.DMA((2, NB)),
            pltpu.SemaphoreType.REGULAR,
            pltpu.SemaphoreType.REGULAR,
            pltpu.SemaphoreType.DMA,
            pltpu.SemaphoreType.DMA,
            pltpu.SemaphoreType.DMA((3,)),
            pltpu.SemaphoreType.DMA((N_DEV,)),
        ],
        compiler_params=pltpu.CompilerParams(
            collective_id=0,
            vmem_limit_bytes=55 * 1024 * 1024 + 512 * 1024,
        ),
    )(x, w_mat)


# device time: 163218 ns/iter; 1.0122x vs baseline; 1.0122x over previous
import functools

import jax
import jax.numpy as jnp
from jax import lax
from jax.experimental import pallas as pl
from jax.experimental.pallas import tpu as pltpu

N_DEV = 4
NB = 8


def kernel(x, w_mat):
    m_global, k_per = x.shape
    _, n = w_mat.shape
    m_per = m_global // N_DEV
    h = m_per // 2
    nblk = n // NB

    def body(
        x_hbm,
        w_hbm,
        out_ref,
        comm_r,
        comm_l,
        stage_ra,
        stage_la,
        stage_rb,
        stage_lb,
        xc_hi,
        xc_lo,
        xland,
        w_hi,
        w_lo,
        wland,
        amax_src_ref,
        amax_comm_ref,
        send_r,
        recv_r,
        send_l,
        recv_l,
        credit_r,
        credit_l,
        xdma_sem,
        wdma_sem,
        amax_send_sems,
        amax_recv_sems,
    ):
        my = lax.axis_index("i")
        left = lax.rem(my + N_DEV - 1, N_DEV)
        right = lax.rem(my + 1, N_DEV)

        TOP, BOT = 0, 1

        def load(owner, half):
            cp = pltpu.make_async_copy(
                x_hbm.at[pl.ds(owner * m_per + half * h, h), :],
                xland,
                xdma_sem,
            )
            cp.start()
            return cp

        def xconv(slot):
            f = xland[...]
            hi = f.astype(jnp.bfloat16)
            xc_hi[slot] = hi
            xc_lo[slot] = (f - hi.astype(jnp.float32)).astype(jnp.bfloat16)

        def load2(owner_top, owner_bot):
            load(owner_top, TOP).wait()
            xconv(0)
            load(owner_bot, BOT).wait()
            xconv(1)

        def wpiece(b, piece):
            cols = pl.ds(b * nblk, nblk)
            rows = pl.ds(piece * (k_per // 2), k_per // 2)
            return pltpu.make_async_copy(
                w_hbm.at[rows, cols], wland, wdma_sem
            )

        def wconv(b, pre=None):
            cols = pl.ds(b * nblk, nblk)
            for piece in range(2):
                cp = pre if (pre is not None and piece == 0) else None
                if cp is None:
                    cp = wpiece(b, piece)
                    cp.start()
                cp.wait()
                rows = pl.ds(piece * (k_per // 2), k_per // 2)
                f = wland[...]
                hi = f.astype(jnp.bfloat16)
                w_hi[rows, cols] = hi
                w_lo[rows, cols] = (
                    f - hi.astype(jnp.float32)
                ).astype(jnp.bfloat16)

        c0 = load(lax.rem(my + 3, N_DEV), TOP)
        w00 = wpiece(0, 0)
        w00.start()

        barrier_sem = pltpu.get_barrier_semaphore()
        for nbr in (left, right):
            pl.semaphore_signal(
                barrier_sem,
                inc=1,
                device_id=(nbr,),
                device_id_type=pl.DeviceIdType.MESH,
            )
        pl.semaphore_wait(barrier_sem, 2)

        def mmb(dst, slot, b):
            cols = pl.ds(b * nblk, nblk)
            xh, xl = xc_hi[slot], xc_lo[slot]
            wh, wl = w_hi[:, cols], w_lo[:, cols]
            dst[...] = jnp.dot(xh, wh, preferred_element_type=jnp.float32)
            dst[...] = dst[...] + jnp.dot(
                xh, wl, preferred_element_type=jnp.float32
            )
            dst[...] = dst[...] + jnp.dot(
                xl, wh, preferred_element_type=jnp.float32
            )

        def rdma(src, dst, ssem, rsem, dev):
            return pltpu.make_async_remote_copy(
                src_ref=src,
                dst_ref=dst,
                send_sem=ssem,
                recv_sem=rsem,
                device_id=(dev,),
                device_id_type=pl.DeviceIdType.MESH,
            )

        r0, l0 = [], []

        def send0_r(b):
            mmb(stage_ra.at[b], 0, b)
            rb = rdma(
                stage_ra.at[b], comm_r.at[0, b],
                send_r.at[0, b], recv_r.at[0, b], right,
            )
            rb.start()
            r0.append(rb)

        def send0_l(b):
            mmb(stage_la.at[b], 1, b)
            lb = rdma(
                stage_la.at[b], comm_l.at[0, b],
                send_l.at[0, b], recv_l.at[0, b], left,
            )
            lb.start()
            l0.append(lb)

        c0.wait()
        xconv(0)
        c1 = load(lax.rem(my + 1, N_DEV), BOT)
        wconv(0, pre=w00)
        send0_r(0)
        c1.wait()
        xconv(1)
        send0_l(0)
        for b in range(1, NB):
            wconv(b)
            send0_r(b)
            send0_l(b)

        load2(lax.rem(my + 2, N_DEV), lax.rem(my + 2, N_DEV))
        for b in range(NB):
            mmb(stage_rb.at[b], 0, b)
            mmb(stage_lb.at[b], 1, b)

        r1, l1 = [], []
        for b in range(NB):
            r0[b].wait_recv()
            stage_rb[b] = stage_rb[b] + comm_r[0, b]
            rb = rdma(
                stage_rb.at[b], comm_r.at[1, b],
                send_r.at[1, b], recv_r.at[1, b], right,
            )
            rb.start()
            r1.append(rb)
            l0[b].wait_recv()
            stage_lb[b] = stage_lb[b] + comm_l[0, b]
            lb = rdma(
                stage_lb.at[b], comm_l.at[1, b],
                send_l.at[1, b], recv_l.at[1, b], left,
            )
            lb.start()
            l1.append(lb)
        pl.semaphore_signal(
            credit_r, inc=1, device_id=(left,),
            device_id_type=pl.DeviceIdType.MESH,
        )
        pl.semaphore_signal(
            credit_l, inc=1, device_id=(right,),
            device_id_type=pl.DeviceIdType.MESH,
        )

        for b in range(NB):
            r0[b].wait_send()
            l0[b].wait_send()
        load2(lax.rem(my + 1, N_DEV), lax.rem(my + 3, N_DEV))
        for b in range(NB):
            mmb(stage_ra.at[b], 0, b)
            mmb(stage_la.at[b], 1, b)

        pl.semaphore_wait(credit_r, 1)
        pl.semaphore_wait(credit_l, 1)
        r2, l2 = [], []
        for b in range(NB):
            r1[b].wait_recv()
            stage_ra[b] = stage_ra[b] + comm_r[1, b]
            rb = rdma(
                stage_ra.at[b], comm_r.at[0, b],
                send_r.at[2, b], recv_r.at[0, b], right,
            )
            rb.start()
            r2.append(rb)
            l1[b].wait_recv()
            stage_la[b] = stage_la[b] + comm_l[1, b]
            lb = rdma(
                stage_la.at[b], comm_l.at[0, b],
                send_l.at[2, b], recv_l.at[0, b], left,
            )
            lb.start()
            l2.append(lb)

        load2(my, my)
        out_top = out_ref.at[pl.ds(0, h)]
        out_bot = out_ref.at[pl.ds(h, h)]
        for b in range(NB):
            cols = pl.ds(b * nblk, nblk)
            mmb(out_top.at[:, cols], 0, b)
            mmb(out_bot.at[:, cols], 1, b)

        local_max = jnp.float32(0.0)
        for b in range(NB):
            cols = pl.ds(b * nblk, nblk)
            r2[b].wait_recv()
            yt = jnp.maximum(out_top[:, cols] + comm_r[0, b], 0.0)
            out_top[:, cols] = yt
            local_max = jnp.maximum(local_max, jnp.max(yt))
            l2[b].wait_recv()
            yb = jnp.maximum(out_bot[:, cols] + comm_l[0, b], 0.0)
            out_bot[:, cols] = yb
            local_max = jnp.maximum(local_max, jnp.max(yb))

        amax_src_ref[...] = jnp.full((8, 128), local_max, jnp.float32)
        amax_comm_ref[my] = amax_src_ref[...]
        peers = [lax.rem(my + d, N_DEV) for d in (1, 2, 3)]
        sends = []
        for j, q in enumerate(peers):
            s = rdma(
                amax_src_ref,
                amax_comm_ref.at[my],
                amax_send_sems.at[j],
                amax_recv_sems.at[my],
                q,
            )
            s.start()
            sends.append(s)
        for q in peers:
            rdma(
                amax_src_ref,
                amax_comm_ref.at[q],
                amax_send_sems.at[0],
                amax_recv_sems.at[q],
                right,
            ).wait_recv()
        gmax = jnp.max(amax_comm_ref[...])

        scale = gmax / 448.0
        inv_scale = 448.0 / gmax
        for b in range(NB):
            cols = pl.ds(b * nblk, nblk)
            q8 = jnp.minimum(out_ref[:, cols] * inv_scale, 448.0).astype(
                jnp.float8_e4m3fn
            )
            out_ref[:, cols] = q8.astype(jnp.float32) * scale

        for d in sends:
            d.wait_send()
        for b in range(NB):
            r1[b].wait_send()
            l1[b].wait_send()
            r2[b].wait_send()
            l2[b].wait_send()


    return pl.pallas_call(
        body,
        out_shape=jax.ShapeDtypeStruct((m_per, n), jnp.float32),
        in_specs=[
            pl.BlockSpec(memory_space=pl.ANY),
            pl.BlockSpec(memory_space=pl.ANY),
        ],
        out_specs=pl.BlockSpec(memory_space=pltpu.VMEM),
        scratch_shapes=[
            pltpu.VMEM((2, NB, h, n // NB), jnp.float32),
            pltpu.VMEM((2, NB, h, n // NB), jnp.float32),
            pltpu.VMEM((NB, h, n // NB), jnp.float32),
            pltpu.VMEM((NB, h, n // NB), jnp.float32),
            pltpu.VMEM((NB, h, n // NB), jnp.float32),
            pltpu.VMEM((NB, h, n // NB), jnp.float32),
            pltpu.VMEM((2, h, k_per), jnp.bfloat16),
            pltpu.VMEM((2, h, k_per), jnp.bfloat16),
            pltpu.VMEM((h, k_per), jnp.float32),
            pltpu.VMEM((k_per, n), jnp.bfloat16),
            pltpu.VMEM((k_per, n), jnp.bfloat16),
            pltpu.VMEM((k_per // 2, n // NB), jnp.float32),
            pltpu.VMEM((8, 128), jnp.float32),
            pltpu.VMEM((N_DEV, 8, 128), jnp.float32),
            pltpu.SemaphoreType.DMA((3, NB)),
            pltpu.SemaphoreType.DMA((2, NB)),
            pltpu.SemaphoreType.DMA((3, NB)),
            pltpu.SemaphoreType.DMA((2, NB)),
            pltpu.SemaphoreType.REGULAR,
            pltpu.SemaphoreType.REGULAR,
            pltpu.SemaphoreType.DMA,
            pltpu.SemaphoreType.DMA,
            pltpu.SemaphoreType.DMA((3,)),
            pltpu.SemaphoreType.DMA((N_DEV,)),
        ],
        compiler_params=pltpu.CompilerParams(
            collective_id=0,
            vmem_limit_bytes=55 * 1024 * 1024 + 512 * 1024,
        ),
    )(x, w_mat)
